# baseline (speedup 1.0000x reference)
import functools

import jax
import jax.numpy as jnp
from jax import lax
from jax.experimental import pallas as pl
from jax.experimental.pallas import tpu as pltpu
from jax.experimental.pallas import tpu_sc as plsc

_NC = 2
_NS = 16
_NW = _NC * _NS
_L = 16


@functools.cache
def _build_sc_embed(B, D, eps):
    rows_per_w = B // _NW
    CHUNK = 64
    n_chunks = rows_per_w // CHUNK
    n_vec = D // _L

    mesh = plsc.VectorSubcoreMesh(core_axis_name="c", subcore_axis_name="s")

    def hsum(v):
        s = v[0]
        for i in range(1, _L):
            s = s + v[i]
        return s

    @functools.partial(
        pl.kernel,
        out_type=jax.ShapeDtypeStruct((B, D), jnp.float32),
        mesh=mesh,
        scratch_types=[
            pltpu.VMEM((rows_per_w,), jnp.int32),
            pltpu.VMEM((rows_per_w + _L,), jnp.int32),
            pltpu.VMEM((CHUNK, D), jnp.float32),
            pltpu.VMEM((CHUNK // 4, D), jnp.float32),
            pltpu.VMEM((2, D), jnp.float32),
            pltpu.VMEM((D,), jnp.float32),
            pltpu.VMEM((D,), jnp.float32),
            pltpu.SemaphoreType.DMA,
        ],
    )
    def sc_embed(src_ref, tt_ref, emb_ref, pos_ref, ttab_ref, g_ref, b_ref,
                 out_ref, idx_v, ttv_v, x_buf, pos_buf, ttab_v, g_v, b_v, sem):
        wid = lax.axis_index("s") * _NC + lax.axis_index("c")
        base = wid * rows_per_w

        pltpu.sync_copy(src_ref.at[pl.ds(base, rows_per_w)], idx_v)
        pltpu.sync_copy(tt_ref.at[pl.ds(base, rows_per_w)],
                        ttv_v.at[pl.ds(0, rows_per_w)])
        pltpu.sync_copy(ttab_ref, ttab_v)
        pltpu.sync_copy(g_ref, g_v)
        pltpu.sync_copy(b_ref, b_v)

        def chunk_body(c, carry):
            row0 = pl.multiple_of(base + c * CHUNK, CHUNK)
            s0 = pl.multiple_of(row0 // 4, CHUNK // 4)
            pltpu.async_copy(
                emb_ref.at[idx_v.at[pl.ds(c * CHUNK, CHUNK)]], x_buf, sem
            ).wait()
            pltpu.sync_copy(pos_ref.at[pl.ds(s0, CHUNK // 4)], pos_buf)

            def row_body(r, rc):
                sl = lax.shift_right_logical(r, 2)
                tts = ttv_v[pl.ds(c * CHUNK + r, _L)][0]
                w16 = jnp.full((_L,), tts.astype(jnp.float32))
                acc_s = jnp.zeros((_L,), jnp.float32)
                acc_q = jnp.zeros((_L,), jnp.float32)
                for j in range(n_vec):
                    sl16 = pl.ds(j * _L, _L)
                    t0 = ttab_v[0, sl16]
                    t1 = ttab_v[1, sl16]
                    x = (x_buf[r, sl16] + pos_buf[sl, sl16]
                         + t0 + w16 * (t1 - t0))
                    x_buf[r, sl16] = x
                    acc_s = acc_s + x
                    acc_q = acc_q + x * x
                mean = hsum(acc_s) * (1.0 / D)
                var = hsum(acc_q) * (1.0 / D) - mean * mean
                vs = var + eps
                bi = 0x5F3759DF - lax.shift_right_logical(
                    lax.bitcast_convert_type(vs, jnp.int32), 1)
                g = lax.bitcast_convert_type(bi, jnp.float32)
                for _ in range(3):
                    g = g * (1.5 - 0.5 * vs * g * g)
                rstd = jnp.full((_L,), g, jnp.float32)
                mean16 = jnp.full((_L,), mean, jnp.float32)
                for j in range(n_vec):
                    sl16 = pl.ds(j * _L, _L)
                    x_buf[r, sl16] = ((x_buf[r, sl16] - mean16) * rstd
                                      * g_v[sl16] + b_v[sl16])
                return rc

            lax.fori_loop(0, CHUNK, row_body, 0)
            pltpu.sync_copy(x_buf, out_ref.at[pl.ds(row0, CHUNK)])
            return carry

        lax.fori_loop(0, n_chunks, chunk_body, 0)

    return sc_embed


def kernel(src, token_type_input, embed_table, pos_table, tok_type_table,
           ln_gamma, ln_beta):
    S, N = src.shape
    D = embed_table.shape[1]
    B = S * N
    sc_embed = _build_sc_embed(B, D, 1e-5)
    out = sc_embed(
        src.reshape(B).astype(jnp.int32),
        token_type_input.reshape(B).astype(jnp.int32),
        embed_table,
        pos_table,
        tok_type_table,
        ln_gamma,
        ln_beta,
    )
    return out.reshape(S, N, D)


# speedup vs baseline: 2.6195x; 2.6195x over previous
import functools

import jax
import jax.numpy as jnp
from jax import lax
from jax.experimental import pallas as pl
from jax.experimental.pallas import tpu as pltpu
from jax.experimental.pallas import tpu_sc as plsc

_NC = 2
_NS = 16
_NW = _NC * _NS


@functools.cache
def _build_sc_gather(B, D):
    rows_per_w = B // _NW
    CHUNK = 32
    n_chunks = rows_per_w // CHUNK
    NBUF = 2

    mesh = plsc.VectorSubcoreMesh(core_axis_name="c", subcore_axis_name="s")

    @functools.partial(
        pl.kernel,
        out_type=jax.ShapeDtypeStruct((B, D), jnp.float32),
        mesh=mesh,
        scratch_types=[
            pltpu.VMEM((rows_per_w,), jnp.int32),
            pltpu.VMEM((NBUF, CHUNK, D), jnp.float32),
            pltpu.SemaphoreType.DMA((NBUF,)),
        ],
    )
    def sc_gather(src_ref, emb_ref, out_ref, idx_v, x_buf, sems):
        wid = lax.axis_index("s") * _NC + lax.axis_index("c")
        base = wid * rows_per_w
        pltpu.sync_copy(src_ref.at[pl.ds(base, rows_per_w)], idx_v)

        descs = [None] * NBUF
        for c in range(n_chunks + 1):
            if c < n_chunks:
                b = c % NBUF
                descs[b] = pltpu.async_copy(
                    emb_ref.at[idx_v.at[pl.ds(c * CHUNK, CHUNK)]],
                    x_buf.at[b], sems.at[b])
            if c >= 1:
                p = (c - 1) % NBUF
                descs[p].wait()
                pltpu.sync_copy(
                    x_buf.at[p],
                    out_ref.at[pl.ds(base + (c - 1) * CHUNK, CHUNK)])

    return sc_gather


@functools.cache
def _build_tc_ln(S, N, D, eps):
    SB = 64
    grid = (S // SB,)

    def tc_ln(tok_ref, tt_ref, pos_ref, ttab_ref, g_ref, b_ref, out_ref):
        x = tok_ref[...]
        x = x + pos_ref[...][:, None, :]
        w = tt_ref[...].astype(jnp.float32)[..., None]
        t0 = ttab_ref[0]
        t1 = ttab_ref[1]
        x = x + t0[None, None, :] + w * (t1 - t0)[None, None, :]
        mean = jnp.mean(x, axis=-1, keepdims=True)
        xc = x - mean
        var = jnp.mean(xc * xc, axis=-1, keepdims=True)
        out_ref[...] = (xc * lax.rsqrt(var + eps) * g_ref[0][None, None, :]
                        + b_ref[0][None, None, :])

    return pl.pallas_call(
        tc_ln,
        grid=grid,
        in_specs=[
            pl.BlockSpec((SB, N, D), lambda i: (i, 0, 0)),
            pl.BlockSpec((SB, N), lambda i: (i, 0)),
            pl.BlockSpec((SB, D), lambda i: (i, 0)),
            pl.BlockSpec((2, D), lambda i: (0, 0)),
            pl.BlockSpec((1, D), lambda i: (0, 0)),
            pl.BlockSpec((1, D), lambda i: (0, 0)),
        ],
        out_specs=pl.BlockSpec((SB, N, D), lambda i: (i, 0, 0)),
        out_shape=jax.ShapeDtypeStruct((S, N, D), jnp.float32),
    )


def kernel(src, token_type_input, embed_table, pos_table, tok_type_table,
           ln_gamma, ln_beta):
    S, N = src.shape
    D = embed_table.shape[1]
    B = S * N
    tok = _build_sc_gather(B, D)(src.reshape(B).astype(jnp.int32),
                                 embed_table)
    out = _build_tc_ln(S, N, D, 1e-5)(
        tok.reshape(S, N, D),
        token_type_input.astype(jnp.int32),
        pos_table,
        tok_type_table,
        ln_gamma.reshape(1, D),
        ln_beta.reshape(1, D),
    )
    return out
